# Initial kernel scaffold; baseline (speedup 1.0000x reference)
import jax
import jax.numpy as jnp
from jax.experimental import pallas as pl


def kernel(x, patch_fc1_w, patch_fc1_b, patch_fc2_w, patch_fc2_b, gate_delta, atom_in_w, atom_in_b, atom_out_w, atom_out_b):
    raise NotImplementedError("write your pallas kernel here")


# speedup vs baseline: 3.8201x; 3.8201x over previous
import functools

import jax
import jax.numpy as jnp
from jax.experimental import pallas as pl
from jax.experimental.pallas import tpu as pltpu

_NCLS = 6
_NUM_ATOMS = 5
_CP = 8


def _gelu(v):
    return 0.5 * v * (1.0 + jax.lax.erf(v * (2.0 ** -0.5)))


def _router(cls_tok, gd_tok):
    t_count = cls_tok.shape[0]
    logits = jnp.sum(cls_tok * gd_tok, axis=1, keepdims=True)
    choose_left = logits >= 0.0
    t = jax.lax.broadcasted_iota(jnp.int32, (t_count, 1), 0)
    n = t % _CP
    valid = n < _NCLS
    l = n // 2
    r = 3 + (n % 2)
    src = jnp.where(valid & choose_left, l, jnp.where(valid, r, -1))
    dst = jnp.where(valid & choose_left, r, jnp.where(valid, l, -1))
    p = jax.nn.sigmoid(logits)
    w = jnp.where(choose_left, p, 1.0 - p)
    return src, dst, w


def _patch_mlp_kernel(x_ref, w1_ref, b1_ref, w2_ref, b2_ref, out_ref,
                      w1b_ref, w2b_ref):
    @pl.when(pl.program_id(0) == 0)
    def _():
        w1b_ref[...] = w1_ref[...].astype(jnp.bfloat16)
        w2b_ref[...] = w2_ref[...].astype(jnp.bfloat16)

    xb = x_ref[...].astype(jnp.bfloat16)
    h = jax.lax.dot_general(
        xb, w1b_ref[...], (((1,), (1,)), ((), ())),
        preferred_element_type=jnp.float32)
    h = _gelu(h + b1_ref[...]).astype(jnp.bfloat16)
    out_ref[...] = jax.lax.dot_general(
        h, w2b_ref[...], (((1,), (1,)), ((), ())),
        preferred_element_type=jnp.float32) + b2_ref[...]


def _cls_hid_kernel(cls_ref, gd_ref, win_ref, bin_ref, hid_ref):
    e = pl.program_id(0)
    src, _, _ = _router(cls_ref[...], gd_ref[...])
    val = jax.lax.dot_general(
        cls_ref[...], win_ref[0], (((1,), (1,)), ((), ())),
        preferred_element_type=jnp.float32)
    val = _gelu(val + bin_ref[0])
    mask = src == e

    @pl.when(e == 0)
    def _():
        hid_ref[...] = jnp.where(mask, val, 0.0)

    @pl.when(e != 0)
    def _():
        hid_ref[...] = jnp.where(mask, val, hid_ref[...])


def _cls_out_kernel(cls_ref, gd_ref, hid_ref, wout_ref, bout_ref, patch_ref,
                    out_ref):
    e = pl.program_id(0)
    _, dst, w = _router(cls_ref[...], gd_ref[...])
    val = jax.lax.dot_general(
        hid_ref[...], wout_ref[0], (((1,), (1,)), ((), ())),
        preferred_element_type=jnp.float32)
    val = (val + bout_ref[0]) * w
    mask = dst == e
    t = cls_ref.shape[0]
    d = cls_ref.shape[1]

    @pl.when(e == 0)
    def _():
        out_ref[...] = jnp.where(mask, val,
                                 patch_ref[...].reshape(t, d)).reshape(
                                     out_ref.shape)

    @pl.when(e != 0)
    def _():
        out_ref[...] = jnp.where(mask, val,
                                 out_ref[...].reshape(t, d)).reshape(
                                     out_ref.shape)


@functools.partial(jax.jit, static_argnames=("interpret",))
def kernel(x, patch_fc1_w, patch_fc1_b, patch_fc2_w, patch_fc2_b,
           gate_delta, atom_in_w, atom_in_b, atom_out_w, atom_out_b,
           interpret=False):
    B, S, D = x.shape
    H = patch_fc1_w.shape[0]
    T = B * _CP

    x_flat = x.reshape(B * S, D)
    n_tok = B * S
    tile = 256
    grid = n_tok // tile
    patch_out = pl.pallas_call(
        _patch_mlp_kernel,
        grid=(grid,),
        in_specs=[
            pl.BlockSpec((tile, D), lambda i: (i, 0)),
            pl.BlockSpec((H, D), lambda i: (0, 0)),
            pl.BlockSpec((1, H), lambda i: (0, 0)),
            pl.BlockSpec((D, H), lambda i: (0, 0)),
            pl.BlockSpec((1, D), lambda i: (0, 0)),
        ],
        out_specs=pl.BlockSpec((tile, D), lambda i: (i, 0)),
        out_shape=jax.ShapeDtypeStruct((n_tok, D), x.dtype),
        scratch_shapes=[
            pltpu.VMEM((H, D), jnp.bfloat16),
            pltpu.VMEM((D, H), jnp.bfloat16),
        ],
        interpret=interpret,
    )(x_flat, patch_fc1_w, patch_fc1_b.reshape(1, H),
      patch_fc2_w, patch_fc2_b.reshape(1, D))

    cls_tok = x[:, :_CP].reshape(T, D)
    gd_tok = jnp.tile(
        jnp.pad(gate_delta, ((0, _CP - _NCLS), (0, 0))), (B, 1))

    cls_hid = pl.pallas_call(
        _cls_hid_kernel,
        grid=(_NUM_ATOMS,),
        in_specs=[
            pl.BlockSpec((T, D), lambda e: (0, 0)),
            pl.BlockSpec((T, D), lambda e: (0, 0)),
            pl.BlockSpec((1, H, D), lambda e: (e, 0, 0)),
            pl.BlockSpec((1, 1, H), lambda e: (e, 0, 0)),
        ],
        out_specs=pl.BlockSpec((T, H), lambda e: (0, 0)),
        out_shape=jax.ShapeDtypeStruct((T, H), x.dtype),
        interpret=interpret,
    )(cls_tok, gd_tok, atom_in_w, atom_in_b.reshape(_NUM_ATOMS, 1, H))

    patch_3d = patch_out.reshape(B, S, D)
    out = pl.pallas_call(
        _cls_out_kernel,
        grid=(_NUM_ATOMS,),
        in_specs=[
            pl.BlockSpec((T, D), lambda e: (0, 0)),
            pl.BlockSpec((T, D), lambda e: (0, 0)),
            pl.BlockSpec((T, H), lambda e: (0, 0)),
            pl.BlockSpec((1, D, H), lambda e: (e, 0, 0)),
            pl.BlockSpec((1, 1, D), lambda e: (e, 0, 0)),
            pl.BlockSpec((B, _CP, D), lambda e: (0, 0, 0)),
        ],
        out_specs=pl.BlockSpec((B, _CP, D), lambda e: (0, 0, 0)),
        out_shape=jax.ShapeDtypeStruct((B, S, D), x.dtype),
        input_output_aliases={5: 0},
        interpret=interpret,
    )(cls_tok, gd_tok, cls_hid, atom_out_w,
      atom_out_b.reshape(_NUM_ATOMS, 1, D), patch_3d)
    return out


# speedup vs baseline: 4.1246x; 1.0797x over previous
import functools

import jax
import jax.numpy as jnp
from jax.experimental import pallas as pl
from jax.experimental.pallas import tpu as pltpu

_NCLS = 6
_NUM_ATOMS = 5
_CP = 8
_TILE = 256
_STAGE0 = 2
_SPACING = 2


def _gelu(v):
    return 0.5 * v * (1.0 + jax.lax.erf(v * (2.0 ** -0.5)))


def _router(cls_tok, gd_tok):
    t_count = cls_tok.shape[0]
    logits = jnp.sum(cls_tok * gd_tok, axis=1, keepdims=True)
    choose_left = logits >= 0.0
    t = jax.lax.broadcasted_iota(jnp.int32, (t_count, 1), 0)
    n = t % _CP
    valid = n < _NCLS
    l = n // 2
    r = 3 + (n % 2)
    src = jnp.where(valid & choose_left, l, jnp.where(valid, r, -1))
    dst = jnp.where(valid & choose_left, r, jnp.where(valid, l, -1))
    p = jax.nn.sigmoid(logits)
    w = jnp.where(choose_left, p, 1.0 - p)
    return src, dst, w


def _expert_bias(e, b_ref):
    oh = (jax.lax.broadcasted_iota(jnp.int32, (1, _NUM_ATOMS), 1)
          == e).astype(jnp.float32)
    return jax.lax.dot_general(
        oh, b_ref[...], (((1,), (0,)), ((), ())),
        preferred_element_type=jnp.float32)


def _fused_kernel(x_ref, w1_ref, b1_ref, w2_ref, b2_ref,
                  cls_ref, gd_ref, ain_ref, binb_ref, aout_ref, boutb_ref,
                  out_ref,
                  w1b_ref, w2b_ref, ein_ref, eout_ref, hid_ref, acc_ref,
                  sin, sout):
    i = pl.program_id(0)
    n_steps = pl.num_programs(0)

    @pl.when(i == 0)
    def _():
        w1b_ref[...] = w1_ref[...].astype(jnp.bfloat16)
        w2b_ref[...] = w2_ref[...].astype(jnp.bfloat16)
        pltpu.make_async_copy(ain_ref.at[0], ein_ref, sin).start()

    xb = x_ref[...].astype(jnp.bfloat16)
    h = jax.lax.dot_general(
        xb, w1b_ref[...], (((1,), (1,)), ((), ())),
        preferred_element_type=jnp.float32)
    h = _gelu(h + b1_ref[...]).astype(jnp.bfloat16)
    res = jax.lax.dot_general(
        h, w2b_ref[...], (((1,), (1,)), ((), ())),
        preferred_element_type=jnp.float32) + b2_ref[...]
    out_ref[...] = res

    stage = (i - _STAGE0) // _SPACING
    is_stage = ((i - _STAGE0) >= 0) & ((i - _STAGE0) % _SPACING == 0) & (
        stage < 2 * _NUM_ATOMS)

    @pl.when(is_stage & (stage < _NUM_ATOMS))
    def _():
        e = stage
        pltpu.make_async_copy(ain_ref.at[e], ein_ref, sin).wait()
        src, _, _ = _router(cls_ref[...], gd_ref[...])
        val = jax.lax.dot_general(
            cls_ref[...], ein_ref[...], (((1,), (1,)), ((), ())),
            preferred_element_type=jnp.float32)
        val = _gelu(val + _expert_bias(e, binb_ref))
        mask = src == e

        @pl.when(e == 0)
        def _():
            hid_ref[...] = jnp.where(mask, val, 0.0)

        @pl.when(e != 0)
        def _():
            hid_ref[...] = jnp.where(mask, val, hid_ref[...])

        @pl.when(e < _NUM_ATOMS - 1)
        def _():
            pltpu.make_async_copy(ain_ref.at[e + 1], ein_ref, sin).start()

        @pl.when(e == _NUM_ATOMS - 1)
        def _():
            pltpu.make_async_copy(aout_ref.at[0], eout_ref, sout).start()

    @pl.when(is_stage & (stage >= _NUM_ATOMS))
    def _():
        e = stage - _NUM_ATOMS
        pltpu.make_async_copy(aout_ref.at[e], eout_ref, sout).wait()
        _, dst, w = _router(cls_ref[...], gd_ref[...])
        val = jax.lax.dot_general(
            hid_ref[...], eout_ref[...], (((1,), (1,)), ((), ())),
            preferred_element_type=jnp.float32)
        val = (val + _expert_bias(e, boutb_ref)) * w
        mask = dst == e

        @pl.when(e == 0)
        def _():
            acc_ref[...] = jnp.where(mask, val, 0.0)

        @pl.when(e != 0)
        def _():
            acc_ref[...] = jnp.where(mask, val, acc_ref[...])

        @pl.when(e < _NUM_ATOMS - 1)
        def _():
            pltpu.make_async_copy(aout_ref.at[e + 1], eout_ref, sout).start()

    @pl.when(i >= n_steps - 4)
    def _():
        b = i - (n_steps - 4)
        rows = acc_ref[pl.ds(b * _CP, _CP), :]
        rowmask = jax.lax.broadcasted_iota(jnp.int32, (_CP, 1), 0) < _NCLS
        out_ref[0:_CP, :] = jnp.where(rowmask, rows, res[0:_CP, :])


def _tile_index(i):
    return jnp.where(i < 28, (i // 7) * 8 + (i % 7) + 1, (i - 28) * 8)


@functools.partial(jax.jit, static_argnames=("interpret",))
def kernel(x, patch_fc1_w, patch_fc1_b, patch_fc2_w, patch_fc2_b,
           gate_delta, atom_in_w, atom_in_b, atom_out_w, atom_out_b,
           interpret=False):
    B, S, D = x.shape
    H = patch_fc1_w.shape[0]
    T = B * _CP

    x_flat = x.reshape(B * S, D)
    n_tok = B * S
    grid = n_tok // _TILE

    cls_tok = x[:, :_CP].reshape(T, D)
    gd_tok = jnp.tile(
        jnp.pad(gate_delta, ((0, _CP - _NCLS), (0, 0))), (B, 1))

    out = pl.pallas_call(
        _fused_kernel,
        grid=(grid,),
        in_specs=[
            pl.BlockSpec((_TILE, D), lambda i: (_tile_index(i), 0)),
            pl.BlockSpec((H, D), lambda i: (0, 0)),
            pl.BlockSpec((1, H), lambda i: (0, 0)),
            pl.BlockSpec((D, H), lambda i: (0, 0)),
            pl.BlockSpec((1, D), lambda i: (0, 0)),
            pl.BlockSpec((T, D), lambda i: (0, 0)),
            pl.BlockSpec((T, D), lambda i: (0, 0)),
            pl.BlockSpec(memory_space=pl.ANY),
            pl.BlockSpec((_NUM_ATOMS, H), lambda i: (0, 0)),
            pl.BlockSpec(memory_space=pl.ANY),
            pl.BlockSpec((_NUM_ATOMS, D), lambda i: (0, 0)),
        ],
        out_specs=pl.BlockSpec((_TILE, D), lambda i: (_tile_index(i), 0)),
        out_shape=jax.ShapeDtypeStruct((n_tok, D), x.dtype),
        scratch_shapes=[
            pltpu.VMEM((H, D), jnp.bfloat16),
            pltpu.VMEM((D, H), jnp.bfloat16),
            pltpu.VMEM((H, D), jnp.float32),
            pltpu.VMEM((D, H), jnp.float32),
            pltpu.VMEM((T, H), jnp.float32),
            pltpu.VMEM((T, D), jnp.float32),
            pltpu.SemaphoreType.DMA,
            pltpu.SemaphoreType.DMA,
        ],
        interpret=interpret,
    )(x_flat, patch_fc1_w, patch_fc1_b.reshape(1, H),
      patch_fc2_w, patch_fc2_b.reshape(1, D),
      cls_tok, gd_tok, atom_in_w, atom_in_b, atom_out_w, atom_out_b)
    return out.reshape(B, S, D)


# speedup vs baseline: 4.1735x; 1.0118x over previous
import functools

import jax
import jax.numpy as jnp
from jax.experimental import pallas as pl
from jax.experimental.pallas import tpu as pltpu

_NCLS = 6
_NUM_ATOMS = 5
_CP = 8
_TILE = 256
_STAGE0 = 2
_SPACING = 2


def _gelu(v):
    return 0.5 * v * (1.0 + jax.lax.erf(v * (2.0 ** -0.5)))


def _router(cls_tok, gd_tok):
    t_count = cls_tok.shape[0]
    logits = jnp.sum(cls_tok * gd_tok, axis=1, keepdims=True)
    choose_left = logits >= 0.0
    t = jax.lax.broadcasted_iota(jnp.int32, (t_count, 1), 0)
    n = t % _CP
    valid = n < _NCLS
    l = n // 2
    r = 3 + (n % 2)
    src = jnp.where(valid & choose_left, l, jnp.where(valid, r, -1))
    dst = jnp.where(valid & choose_left, r, jnp.where(valid, l, -1))
    p = jax.nn.sigmoid(logits)
    w = jnp.where(choose_left, p, 1.0 - p)
    return src, dst, w


_NCHUNK = 4


def _expert_copies(src3, dst, sem, e):
    rows = dst.shape[0] // _NCHUNK
    return [
        pltpu.make_async_copy(
            src3.at[e, pl.ds(c * rows, rows), :],
            dst.at[pl.ds(c * rows, rows), :], sem)
        for c in range(_NCHUNK)
    ]


def _start_expert(src3, dst, sem, e):
    for cp in _expert_copies(src3, dst, sem, e):
        cp.start()


def _wait_expert(src3, dst, sem, e):
    for cp in _expert_copies(src3, dst, sem, e):
        cp.wait()


def _expert_bias(e, b_ref):
    oh = (jax.lax.broadcasted_iota(jnp.int32, (1, _NUM_ATOMS), 1)
          == e).astype(jnp.float32)
    return jax.lax.dot_general(
        oh, b_ref[...], (((1,), (0,)), ((), ())),
        preferred_element_type=jnp.float32)


def _fused_kernel(x_ref, w1_ref, b1_ref, w2_ref, b2_ref,
                  cls_ref, gd_ref, ain_ref, binb_ref, aout_ref, boutb_ref,
                  out_ref,
                  w1b_ref, w2b_ref, ein_ref, eout_ref, hid_ref, acc_ref,
                  sin, sout):
    i = pl.program_id(0)
    n_steps = pl.num_programs(0)

    @pl.when(i == 0)
    def _():
        w1b_ref[...] = w1_ref[...].astype(jnp.bfloat16)
        w2b_ref[...] = w2_ref[...].astype(jnp.bfloat16)
        _start_expert(ain_ref, ein_ref, sin, 0)
        _start_expert(aout_ref, eout_ref, sout, 0)

    xb = x_ref[...].astype(jnp.bfloat16)
    h = jax.lax.dot_general(
        xb, w1b_ref[...], (((1,), (1,)), ((), ())),
        preferred_element_type=jnp.float32)
    h = _gelu((h + b1_ref[...]).astype(jnp.bfloat16))
    res = jax.lax.dot_general(
        h, w2b_ref[...], (((1,), (1,)), ((), ())),
        preferred_element_type=jnp.float32) + b2_ref[...]
    out_ref[...] = res

    stage = (i - _STAGE0) // _SPACING
    is_stage = ((i - _STAGE0) >= 0) & ((i - _STAGE0) % _SPACING == 0) & (
        stage < 2 * _NUM_ATOMS)

    @pl.when(is_stage & (stage < _NUM_ATOMS))
    def _():
        e = stage
        _wait_expert(ain_ref, ein_ref, sin, e)
        src, _, _ = _router(cls_ref[...], gd_ref[...])
        val = jax.lax.dot_general(
            cls_ref[...], ein_ref[...], (((1,), (1,)), ((), ())),
            preferred_element_type=jnp.float32)
        val = _gelu(val + _expert_bias(e, binb_ref))
        mask = src == e

        @pl.when(e == 0)
        def _():
            hid_ref[...] = jnp.where(mask, val, 0.0)

        @pl.when(e != 0)
        def _():
            hid_ref[...] = jnp.where(mask, val, hid_ref[...])

        @pl.when(e < _NUM_ATOMS - 1)
        def _():
            _start_expert(ain_ref, ein_ref, sin, e + 1)


    @pl.when(is_stage & (stage >= _NUM_ATOMS))
    def _():
        e = stage - _NUM_ATOMS
        _wait_expert(aout_ref, eout_ref, sout, e)
        _, dst, w = _router(cls_ref[...], gd_ref[...])
        val = jax.lax.dot_general(
            hid_ref[...], eout_ref[...], (((1,), (1,)), ((), ())),
            preferred_element_type=jnp.float32)
        val = (val + _expert_bias(e, boutb_ref)) * w
        mask = dst == e

        @pl.when(e == 0)
        def _():
            acc_ref[...] = jnp.where(mask, val, 0.0)

        @pl.when(e != 0)
        def _():
            acc_ref[...] = jnp.where(mask, val, acc_ref[...])

        @pl.when(e < _NUM_ATOMS - 1)
        def _():
            _start_expert(aout_ref, eout_ref, sout, e + 1)

    @pl.when(i >= n_steps - 4)
    def _():
        b = i - (n_steps - 4)
        rows = acc_ref[pl.ds(b * _CP, _CP), :]
        rowmask = jax.lax.broadcasted_iota(jnp.int32, (_CP, 1), 0) < _NCLS
        out_ref[0:_CP, :] = jnp.where(rowmask, rows, res[0:_CP, :])


def _tile_index(i):
    return jnp.where(i < 28, (i // 7) * 8 + (i % 7) + 1, (i - 28) * 8)


@functools.partial(jax.jit, static_argnames=("interpret",))
def kernel(x, patch_fc1_w, patch_fc1_b, patch_fc2_w, patch_fc2_b,
           gate_delta, atom_in_w, atom_in_b, atom_out_w, atom_out_b,
           interpret=False):
    B, S, D = x.shape
    H = patch_fc1_w.shape[0]
    T = B * _CP

    x_flat = x.reshape(B * S, D)
    n_tok = B * S
    grid = n_tok // _TILE

    cls_tok = x[:, :_CP].reshape(T, D)
    gd_tok = jnp.tile(
        jnp.pad(gate_delta, ((0, _CP - _NCLS), (0, 0))), (B, 1))

    out = pl.pallas_call(
        _fused_kernel,
        grid=(grid,),
        in_specs=[
            pl.BlockSpec((_TILE, D), lambda i: (_tile_index(i), 0)),
            pl.BlockSpec((H, D), lambda i: (0, 0)),
            pl.BlockSpec((1, H), lambda i: (0, 0)),
            pl.BlockSpec((D, H), lambda i: (0, 0)),
            pl.BlockSpec((1, D), lambda i: (0, 0)),
            pl.BlockSpec((T, D), lambda i: (0, 0)),
            pl.BlockSpec((T, D), lambda i: (0, 0)),
            pl.BlockSpec(memory_space=pl.ANY),
            pl.BlockSpec((_NUM_ATOMS, H), lambda i: (0, 0)),
            pl.BlockSpec(memory_space=pl.ANY),
            pl.BlockSpec((_NUM_ATOMS, D), lambda i: (0, 0)),
        ],
        out_specs=pl.BlockSpec((_TILE, D), lambda i: (_tile_index(i), 0)),
        out_shape=jax.ShapeDtypeStruct((n_tok, D), x.dtype),
        scratch_shapes=[
            pltpu.VMEM((H, D), jnp.bfloat16),
            pltpu.VMEM((D, H), jnp.bfloat16),
            pltpu.VMEM((H, D), jnp.float32),
            pltpu.VMEM((D, H), jnp.float32),
            pltpu.VMEM((T, H), jnp.float32),
            pltpu.VMEM((T, D), jnp.float32),
            pltpu.SemaphoreType.DMA,
            pltpu.SemaphoreType.DMA,
        ],
        interpret=interpret,
    )(x_flat, patch_fc1_w, patch_fc1_b.reshape(1, H),
      patch_fc2_w, patch_fc2_b.reshape(1, D),
      cls_tok, gd_tok, atom_in_w, atom_in_b, atom_out_w, atom_out_b)
    return out.reshape(B, S, D)


# speedup vs baseline: 4.8490x; 1.1619x over previous
import functools

import jax
import jax.numpy as jnp
from jax.experimental import pallas as pl
from jax.experimental.pallas import tpu as pltpu

_NCLS = 6
_NUM_ATOMS = 5
_CP = 8
_TILE = 512
_STAGE0 = 2
_SPACING = 1


def _gelu(v):
    return 0.5 * v * (1.0 + jax.lax.erf(v * (2.0 ** -0.5)))


def _router(cls_tok, gd_tok):
    t_count = cls_tok.shape[0]
    logits = jnp.sum(cls_tok * gd_tok, axis=1, keepdims=True)
    choose_left = logits >= 0.0
    t = jax.lax.broadcasted_iota(jnp.int32, (t_count, 1), 0)
    n = t % _CP
    valid = n < _NCLS
    l = n // 2
    r = 3 + (n % 2)
    src = jnp.where(valid & choose_left, l, jnp.where(valid, r, -1))
    dst = jnp.where(valid & choose_left, r, jnp.where(valid, l, -1))
    p = jax.nn.sigmoid(logits)
    w = jnp.where(choose_left, p, 1.0 - p)
    return src, dst, w


_NCHUNK = 4


def _expert_copies(src3, dst, sem, e):
    rows = dst.shape[0] // _NCHUNK
    return [
        pltpu.make_async_copy(
            src3.at[e, pl.ds(c * rows, rows), :],
            dst.at[pl.ds(c * rows, rows), :], sem)
        for c in range(_NCHUNK)
    ]


def _start_expert(src3, dst, sem, e):
    for cp in _expert_copies(src3, dst, sem, e):
        cp.start()


def _wait_expert(src3, dst, sem, e):
    for cp in _expert_copies(src3, dst, sem, e):
        cp.wait()


def _expert_bias(e, b_ref):
    oh = (jax.lax.broadcasted_iota(jnp.int32, (1, _NUM_ATOMS), 1)
          == e).astype(jnp.float32)
    return jax.lax.dot_general(
        oh, b_ref[...], (((1,), (0,)), ((), ())),
        preferred_element_type=jnp.float32)


def _fused_kernel(x_ref, w1_ref, b1_ref, w2_ref, b2_ref,
                  cls_ref, gd_ref, ain_ref, binb_ref, aout_ref, boutb_ref,
                  out_ref,
                  w1b_ref, w2b_ref, ein_ref, eout_ref, hid_ref, acc_ref,
                  sin, sout):
    i = pl.program_id(0)
    n_steps = pl.num_programs(0)

    @pl.when(i == 0)
    def _():
        w1b_ref[...] = w1_ref[...].astype(jnp.bfloat16)
        w2b_ref[...] = w2_ref[...].astype(jnp.bfloat16)
        _start_expert(ain_ref, ein_ref, sin, 0)
        _start_expert(aout_ref, eout_ref, sout, 0)

    xb = x_ref[...].astype(jnp.bfloat16)
    h = jax.lax.dot_general(
        xb, w1b_ref[...], (((1,), (1,)), ((), ())),
        preferred_element_type=jnp.float32)
    h = _gelu((h + b1_ref[...]).astype(jnp.bfloat16))
    res = jax.lax.dot_general(
        h, w2b_ref[...], (((1,), (1,)), ((), ())),
        preferred_element_type=jnp.float32) + b2_ref[...]
    out_ref[...] = res

    stage = (i - _STAGE0) // _SPACING
    is_stage = ((i - _STAGE0) >= 0) & ((i - _STAGE0) % _SPACING == 0) & (
        stage < 2 * _NUM_ATOMS)

    @pl.when(is_stage & (stage < _NUM_ATOMS))
    def _():
        e = stage
        _wait_expert(ain_ref, ein_ref, sin, e)
        src, _, _ = _router(cls_ref[...], gd_ref[...])
        val = jax.lax.dot_general(
            cls_ref[...], ein_ref[...], (((1,), (1,)), ((), ())),
            preferred_element_type=jnp.float32)
        val = _gelu(val + _expert_bias(e, binb_ref))
        mask = src == e

        @pl.when(e == 0)
        def _():
            hid_ref[...] = jnp.where(mask, val, 0.0)

        @pl.when(e != 0)
        def _():
            hid_ref[...] = jnp.where(mask, val, hid_ref[...])

        @pl.when(e < _NUM_ATOMS - 1)
        def _():
            _start_expert(ain_ref, ein_ref, sin, e + 1)


    @pl.when(is_stage & (stage >= _NUM_ATOMS))
    def _():
        e = stage - _NUM_ATOMS
        _wait_expert(aout_ref, eout_ref, sout, e)
        _, dst, w = _router(cls_ref[...], gd_ref[...])
        val = jax.lax.dot_general(
            hid_ref[...], eout_ref[...], (((1,), (1,)), ((), ())),
            preferred_element_type=jnp.float32)
        val = (val + _expert_bias(e, boutb_ref)) * w
        mask = dst == e

        @pl.when(e == 0)
        def _():
            acc_ref[...] = jnp.where(mask, val, 0.0)

        @pl.when(e != 0)
        def _():
            acc_ref[...] = jnp.where(mask, val, acc_ref[...])

        @pl.when(e < _NUM_ATOMS - 1)
        def _():
            _start_expert(aout_ref, eout_ref, sout, e + 1)

    @pl.when(i >= n_steps - 4)
    def _():
        b = i - (n_steps - 4)
        rows = acc_ref[pl.ds(b * _CP, _CP), :]
        rowmask = jax.lax.broadcasted_iota(jnp.int32, (_CP, 1), 0) < _NCLS
        out_ref[0:_CP, :] = jnp.where(rowmask, rows, res[0:_CP, :])


def _tile_index(i):
    k = 2048 // _TILE
    nc = k - 1
    return jnp.where(i < 4 * nc, (i // nc) * k + (i % nc) + 1,
                     (i - 4 * nc) * k)


@functools.partial(jax.jit, static_argnames=("interpret",))
def kernel(x, patch_fc1_w, patch_fc1_b, patch_fc2_w, patch_fc2_b,
           gate_delta, atom_in_w, atom_in_b, atom_out_w, atom_out_b,
           interpret=False):
    B, S, D = x.shape
    H = patch_fc1_w.shape[0]
    T = B * _CP

    x_flat = x.reshape(B * S, D)
    n_tok = B * S
    grid = n_tok // _TILE

    cls_tok = x[:, :_CP].reshape(T, D)
    gd_tok = jnp.tile(
        jnp.pad(gate_delta, ((0, _CP - _NCLS), (0, 0))), (B, 1))

    out = pl.pallas_call(
        _fused_kernel,
        grid=(grid,),
        in_specs=[
            pl.BlockSpec((_TILE, D), lambda i: (_tile_index(i), 0)),
            pl.BlockSpec((H, D), lambda i: (0, 0)),
            pl.BlockSpec((1, H), lambda i: (0, 0)),
            pl.BlockSpec((D, H), lambda i: (0, 0)),
            pl.BlockSpec((1, D), lambda i: (0, 0)),
            pl.BlockSpec((T, D), lambda i: (0, 0)),
            pl.BlockSpec((T, D), lambda i: (0, 0)),
            pl.BlockSpec(memory_space=pl.ANY),
            pl.BlockSpec((_NUM_ATOMS, H), lambda i: (0, 0)),
            pl.BlockSpec(memory_space=pl.ANY),
            pl.BlockSpec((_NUM_ATOMS, D), lambda i: (0, 0)),
        ],
        out_specs=pl.BlockSpec((_TILE, D), lambda i: (_tile_index(i), 0)),
        out_shape=jax.ShapeDtypeStruct((n_tok, D), x.dtype),
        scratch_shapes=[
            pltpu.VMEM((H, D), jnp.bfloat16),
            pltpu.VMEM((D, H), jnp.bfloat16),
            pltpu.VMEM((H, D), jnp.float32),
            pltpu.VMEM((D, H), jnp.float32),
            pltpu.VMEM((T, H), jnp.float32),
            pltpu.VMEM((T, D), jnp.float32),
            pltpu.SemaphoreType.DMA,
            pltpu.SemaphoreType.DMA,
        ],
        interpret=interpret,
    )(x_flat, patch_fc1_w, patch_fc1_b.reshape(1, H),
      patch_fc2_w, patch_fc2_b.reshape(1, D),
      cls_tok, gd_tok, atom_in_w, atom_in_b, atom_out_w, atom_out_b)
    return out.reshape(B, S, D)


# speedup vs baseline: 4.8548x; 1.0012x over previous
import functools

import jax
import jax.numpy as jnp
from jax.experimental import pallas as pl
from jax.experimental.pallas import tpu as pltpu

_NCLS = 6
_NUM_ATOMS = 5
_CP = 8
_TILE = 512
_STAGE0 = 1
_SPACING = 1


def _gelu(v):
    return 0.5 * v * (1.0 + jax.lax.erf(v * (2.0 ** -0.5)))


def _router(cls_tok, gd_tok):
    t_count = cls_tok.shape[0]
    logits = jnp.sum(cls_tok * gd_tok, axis=1, keepdims=True)
    choose_left = logits >= 0.0
    t = jax.lax.broadcasted_iota(jnp.int32, (t_count, 1), 0)
    n = t % _CP
    valid = n < _NCLS
    l = n // 2
    r = 3 + (n % 2)
    src = jnp.where(valid & choose_left, l, jnp.where(valid, r, -1))
    dst = jnp.where(valid & choose_left, r, jnp.where(valid, l, -1))
    p = jax.nn.sigmoid(logits)
    w = jnp.where(choose_left, p, 1.0 - p)
    return src, dst, w


_NCHUNK = 8


def _expert_copies(src3, dst, sem, e):
    rows = dst.shape[0] // _NCHUNK
    return [
        pltpu.make_async_copy(
            src3.at[e, pl.ds(c * rows, rows), :],
            dst.at[pl.ds(c * rows, rows), :], sem)
        for c in range(_NCHUNK)
    ]


def _start_expert(src3, dst, sem, e):
    for cp in _expert_copies(src3, dst, sem, e):
        cp.start()


def _wait_expert(src3, dst, sem, e):
    for cp in _expert_copies(src3, dst, sem, e):
        cp.wait()


def _expert_bias(e, b_ref):
    oh = (jax.lax.broadcasted_iota(jnp.int32, (1, _NUM_ATOMS), 1)
          == e).astype(jnp.float32)
    return jax.lax.dot_general(
        oh, b_ref[...], (((1,), (0,)), ((), ())),
        preferred_element_type=jnp.float32)


def _fused_kernel(x_ref, w1_ref, b1_ref, w2_ref, b2_ref,
                  cls_ref, gd_ref, ain_ref, binb_ref, aout_ref, boutb_ref,
                  out_ref,
                  w1b_ref, w2b_ref, ein_ref, eout_ref, hid_ref, acc_ref,
                  sin, sout):
    i = pl.program_id(0)
    n_steps = pl.num_programs(0)

    @pl.when(i == 0)
    def _():
        w1b_ref[...] = w1_ref[...].astype(jnp.bfloat16)
        w2b_ref[...] = w2_ref[...].astype(jnp.bfloat16)
        _start_expert(ain_ref, ein_ref, sin, 0)
        _start_expert(aout_ref, eout_ref, sout, 0)

    xb = x_ref[...].astype(jnp.bfloat16)
    h = jax.lax.dot_general(
        xb, w1b_ref[...], (((1,), (1,)), ((), ())),
        preferred_element_type=jnp.float32)
    h = _gelu((h + b1_ref[...]).astype(jnp.bfloat16))
    res = jax.lax.dot_general(
        h, w2b_ref[...], (((1,), (1,)), ((), ())),
        preferred_element_type=jnp.float32) + b2_ref[...]
    out_ref[...] = res

    stage = (i - _STAGE0) // _SPACING
    is_stage = ((i - _STAGE0) >= 0) & ((i - _STAGE0) % _SPACING == 0) & (
        stage < 2 * _NUM_ATOMS)

    @pl.when(is_stage & (stage < _NUM_ATOMS))
    def _():
        e = stage
        _wait_expert(ain_ref, ein_ref, sin, e)
        src, _, _ = _router(cls_ref[...], gd_ref[...])
        val = jax.lax.dot_general(
            cls_ref[...], ein_ref[...], (((1,), (1,)), ((), ())),
            preferred_element_type=jnp.float32)
        val = _gelu(val + _expert_bias(e, binb_ref))
        mask = src == e

        @pl.when(e == 0)
        def _():
            hid_ref[...] = jnp.where(mask, val, 0.0)

        @pl.when(e != 0)
        def _():
            hid_ref[...] = jnp.where(mask, val, hid_ref[...])

        @pl.when(e < _NUM_ATOMS - 1)
        def _():
            _start_expert(ain_ref, ein_ref, sin, e + 1)


    @pl.when(is_stage & (stage >= _NUM_ATOMS))
    def _():
        e = stage - _NUM_ATOMS
        _wait_expert(aout_ref, eout_ref, sout, e)
        _, dst, w = _router(cls_ref[...], gd_ref[...])
        val = jax.lax.dot_general(
            hid_ref[...], eout_ref[...], (((1,), (1,)), ((), ())),
            preferred_element_type=jnp.float32)
        val = (val + _expert_bias(e, boutb_ref)) * w
        mask = dst == e

        @pl.when(e == 0)
        def _():
            acc_ref[...] = jnp.where(mask, val, 0.0)

        @pl.when(e != 0)
        def _():
            acc_ref[...] = jnp.where(mask, val, acc_ref[...])

        @pl.when(e < _NUM_ATOMS - 1)
        def _():
            _start_expert(aout_ref, eout_ref, sout, e + 1)

    @pl.when(i >= n_steps - 4)
    def _():
        b = i - (n_steps - 4)
        rows = acc_ref[pl.ds(b * _CP, _CP), :]
        rowmask = jax.lax.broadcasted_iota(jnp.int32, (_CP, 1), 0) < _NCLS
        out_ref[0:_CP, :] = jnp.where(rowmask, rows, res[0:_CP, :])


def _tile_index(i):
    k = 2048 // _TILE
    nc = k - 1
    return jnp.where(i < 4 * nc, (i // nc) * k + (i % nc) + 1,
                     (i - 4 * nc) * k)


@functools.partial(jax.jit, static_argnames=("interpret",))
def kernel(x, patch_fc1_w, patch_fc1_b, patch_fc2_w, patch_fc2_b,
           gate_delta, atom_in_w, atom_in_b, atom_out_w, atom_out_b,
           interpret=False):
    B, S, D = x.shape
    H = patch_fc1_w.shape[0]
    T = B * _CP

    x_flat = x.reshape(B * S, D)
    n_tok = B * S
    grid = n_tok // _TILE

    cls_tok = x[:, :_CP].reshape(T, D)
    gd_tok = jnp.tile(
        jnp.pad(gate_delta, ((0, _CP - _NCLS), (0, 0))), (B, 1))

    out = pl.pallas_call(
        _fused_kernel,
        grid=(grid,),
        in_specs=[
            pl.BlockSpec((_TILE, D), lambda i: (_tile_index(i), 0)),
            pl.BlockSpec((H, D), lambda i: (0, 0)),
            pl.BlockSpec((1, H), lambda i: (0, 0)),
            pl.BlockSpec((D, H), lambda i: (0, 0)),
            pl.BlockSpec((1, D), lambda i: (0, 0)),
            pl.BlockSpec((T, D), lambda i: (0, 0)),
            pl.BlockSpec((T, D), lambda i: (0, 0)),
            pl.BlockSpec(memory_space=pl.ANY),
            pl.BlockSpec((_NUM_ATOMS, H), lambda i: (0, 0)),
            pl.BlockSpec(memory_space=pl.ANY),
            pl.BlockSpec((_NUM_ATOMS, D), lambda i: (0, 0)),
        ],
        out_specs=pl.BlockSpec((_TILE, D), lambda i: (_tile_index(i), 0)),
        out_shape=jax.ShapeDtypeStruct((n_tok, D), x.dtype),
        scratch_shapes=[
            pltpu.VMEM((H, D), jnp.bfloat16),
            pltpu.VMEM((D, H), jnp.bfloat16),
            pltpu.VMEM((H, D), jnp.float32),
            pltpu.VMEM((D, H), jnp.float32),
            pltpu.VMEM((T, H), jnp.float32),
            pltpu.VMEM((T, D), jnp.float32),
            pltpu.SemaphoreType.DMA,
            pltpu.SemaphoreType.DMA,
        ],
        interpret=interpret,
    )(x_flat, patch_fc1_w, patch_fc1_b.reshape(1, H),
      patch_fc2_w, patch_fc2_b.reshape(1, D),
      cls_tok, gd_tok, atom_in_w, atom_in_b, atom_out_w, atom_out_b)
    return out.reshape(B, S, D)
